# baseline (speedup 1.0000x reference)
import functools

import jax
import jax.numpy as jnp
from jax.experimental import pallas as pl
from jax.experimental.pallas import tpu as pltpu

PATCH = 16
BINS = 8
NSTREAM = 4


def _fused_kernel(t0, t1, t2, t3, p0, p1, p2, p3, m_ref, loss_ref, msum_ref):
    i = pl.program_id(0)
    j = pl.program_id(1)
    t_refs = (t0, t1, t2, t3)
    p_refs = (p0, p1, p2, p3)
    H = t0.shape[2]
    W = t0.shape[3]
    h = H // PATCH
    w = W // PATCH
    nclass = p0.shape[-1]
    rT = jax.lax.broadcasted_iota(jnp.int32, (W, w), 0) // PATCH
    cT = jax.lax.broadcasted_iota(jnp.int32, (W, w), 1)
    PT = (rT == cT).astype(jnp.float32)

    part = jnp.zeros((1, 1, 1, 1), jnp.float32)
    pm = jnp.zeros((1, 1, 1, 1), jnp.float32)
    for k in range(NSTREAM):
        label = jnp.zeros((h, w), dtype=jnp.int32)
        for ch in range(3):
            tc = jnp.minimum(t_refs[k][0, ch], 1.0)
            rs = jnp.sum(tc.reshape(h, PATCH, W), axis=1)
            psum = jax.lax.dot(rs, PT, precision=jax.lax.Precision.HIGHEST,
                               preferred_element_type=jnp.float32)
            d = jnp.zeros((h, w), dtype=jnp.int32)
            for kb in range(1, BINS):
                d += (psum > (kb * PATCH * PATCH / BINS)).astype(jnp.int32)
            label += d * (BINS ** ch)
        mlab = jnp.where(m_ref[k, i, pl.ds(j * h, h)] != 0, label, -1)

        p3d = p_refs[k][...].reshape(h, w, nclass)
        s = jnp.sum(jnp.exp(p3d), axis=2)
        lse = jnp.log(s)
        oh = jax.lax.broadcasted_iota(jnp.int32, p3d.shape, 2) == mlab[:, :, None]
        corr = jnp.sum(jnp.where(oh, p3d, 0.0), axis=2)
        m2 = (mlab >= 0).astype(jnp.float32)
        part += jnp.sum(m2 * (lse - corr)).reshape(1, 1, 1, 1)
        pm += jnp.sum(m2).reshape(1, 1, 1, 1)

    loss_ref[...] = part
    msum_ref[...] = pm


@functools.partial(jax.jit, static_argnames=())
def kernel(predicted, target, mask):
    B, C, H, W = target.shape
    h = H // PATCH
    w = W // PATCH
    n_patches = h * w
    ns = NSTREAM
    bq = B // ns

    maski = mask.astype(jnp.int32).reshape(ns, bq, h, w)
    pred2d = predicted.reshape(B * n_patches, predicted.shape[-1])
    nclass = pred2d.shape[-1]

    t_spec = [
        pl.BlockSpec((1, C, H // 2, W),
                     (lambda k: (lambda i, j: (k * bq + i, 0, j, 0)))(k))
        for k in range(ns)
    ]
    p_spec = [
        pl.BlockSpec((n_patches // 2, nclass),
                     (lambda k: (lambda i, j: (2 * (k * bq + i) + j, 0)))(k))
        for k in range(ns)
    ]
    sums = pl.pallas_call(
        _fused_kernel,
        grid=(bq, 2),
        in_specs=t_spec + p_spec
        + [pl.BlockSpec((ns, bq, h, w), lambda i, j: (0, 0, 0, 0))],
        out_specs=[
            pl.BlockSpec((1, 1, 1, 1), lambda i, j: (i, j, 0, 0)),
            pl.BlockSpec((1, 1, 1, 1), lambda i, j: (i, j, 0, 0)),
        ],
        out_shape=[
            jax.ShapeDtypeStruct((bq, 2, 1, 1), jnp.float32),
            jax.ShapeDtypeStruct((bq, 2, 1, 1), jnp.float32),
        ],
        compiler_params=pltpu.CompilerParams(
            dimension_semantics=("parallel", "arbitrary")),
    )(*([target] * ns), *([pred2d] * ns), maski)

    return jnp.sum(sums[0]) / jnp.sum(sums[1])


# speedup vs baseline: 1.0529x; 1.0529x over previous
import functools

import jax
import jax.numpy as jnp
from jax.experimental import pallas as pl
from jax.experimental.pallas import tpu as pltpu

PATCH = 16
BINS = 8
NSTREAM = 4


def _fused_kernel(t0, t1, t2, t3, p0, p1, p2, p3, m_ref, loss_ref, msum_ref):
    i = pl.program_id(0)
    t_refs = (t0, t1, t2, t3)
    p_refs = (p0, p1, p2, p3)
    H = t0.shape[2]
    W = t0.shape[3]
    h = H // PATCH
    w = W // PATCH
    nclass = p0.shape[-1]
    rT = jax.lax.broadcasted_iota(jnp.int32, (W, w), 0) // PATCH
    cT = jax.lax.broadcasted_iota(jnp.int32, (W, w), 1)
    PT = (rT == cT).astype(jnp.float32)

    part = jnp.zeros((1, 1, 1), jnp.float32)
    pm = jnp.zeros((1, 1, 1), jnp.float32)
    for k in range(NSTREAM):
        label = jnp.zeros((h, w), dtype=jnp.int32)
        for ch in range(3):
            tc = jnp.minimum(t_refs[k][0, ch], 1.0)
            rs = jnp.sum(tc.reshape(h, PATCH, W), axis=1)
            psum = jax.lax.dot(rs, PT, precision=jax.lax.Precision.HIGHEST,
                               preferred_element_type=jnp.float32)
            d = jnp.zeros((h, w), dtype=jnp.int32)
            for kb in range(1, BINS):
                d += (psum > (kb * PATCH * PATCH / BINS)).astype(jnp.int32)
            label += d * (BINS ** ch)
        mlab = jnp.where(m_ref[k, i] != 0, label, -1)

        p3d = p_refs[k][...].reshape(h, w, nclass)
        s = jnp.sum(jnp.exp(p3d), axis=2)
        lse = jnp.log(s)
        oh = jax.lax.broadcasted_iota(jnp.int32, p3d.shape, 2) == mlab[:, :, None]
        corr = jnp.sum(jnp.where(oh, p3d, 0.0), axis=2)
        m2 = (mlab >= 0).astype(jnp.float32)
        part += jnp.sum(m2 * (lse - corr)).reshape(1, 1, 1)
        pm += jnp.sum(m2).reshape(1, 1, 1)

    loss_ref[...] = part
    msum_ref[...] = pm


@functools.partial(jax.jit, static_argnames=())
def kernel(predicted, target, mask):
    B, C, H, W = target.shape
    h = H // PATCH
    w = W // PATCH
    n_patches = h * w
    ns = NSTREAM
    bq = B // ns

    maski = mask.astype(jnp.int32).reshape(ns, bq, h, w)
    pred2d = predicted.reshape(B * n_patches, predicted.shape[-1])
    nclass = pred2d.shape[-1]

    t_spec = [
        pl.BlockSpec((1, C, H, W), (lambda k: (lambda i: (k * bq + i, 0, 0, 0)))(k))
        for k in range(ns)
    ]
    p_spec = [
        pl.BlockSpec((n_patches, nclass),
                     (lambda k: (lambda i: (k * bq + i, 0)))(k))
        for k in range(ns)
    ]
    sums = pl.pallas_call(
        _fused_kernel,
        grid=(bq,),
        in_specs=t_spec + p_spec
        + [pl.BlockSpec((ns, bq, h, w), lambda i: (0, 0, 0, 0))],
        out_specs=[
            pl.BlockSpec((1, 1, 1), lambda i: (i, 0, 0)),
            pl.BlockSpec((1, 1, 1), lambda i: (i, 0, 0)),
        ],
        out_shape=[
            jax.ShapeDtypeStruct((bq, 1, 1), jnp.float32),
            jax.ShapeDtypeStruct((bq, 1, 1), jnp.float32),
        ],
        compiler_params=pltpu.CompilerParams(
            dimension_semantics=("parallel",)),
    )(*([target] * ns), *([pred2d] * ns), maski)

    return jnp.sum(sums[0]) / jnp.sum(sums[1])
